# baseline (speedup 1.0000x reference)
import functools
from math import log

import jax
import jax.numpy as jnp
from jax import lax
from jax.experimental import pallas as pl
from jax.experimental.pallas import tpu as pltpu, tpu_sc as plsc

NUM_NODES = 100000
EMB_DIM = 128
NEGATIVE = 5
LR = 0.025

NC, NS, L = 2, 16, 16
R = 12800
NRANGE = 8
SUB_ROWS = R // NS
BLK = 2048
C = 64
ACC_ROWS = R + 8
LUT_N = 1216

BIAS_P = float(log(NUM_NODES))
BIAS_N = float(log(NUM_NODES / NEGATIVE))


def _lookup_table():
    t = jax.nn.sigmoid(jnp.arange(-6.01, 6.01, 0.01, dtype=jnp.float32))
    t = t.at[0].set(0.0)
    t = t.at[-1].set(1.0)
    return jnp.concatenate([t, jnp.full((LUT_N - t.shape[0],), 1.0, jnp.float32)])


@functools.lru_cache(maxsize=None)
def _build(total_updates: int):
    S = total_updates // NS

    mesh = plsc.VectorSubcoreMesh(
        core_axis_name="c", subcore_axis_name="s", num_cores=NC, num_subcores=NS
    )

    @functools.partial(
        pl.kernel,
        out_type=jax.ShapeDtypeStruct((NUM_NODES, EMB_DIM), jnp.float32),
        mesh=mesh,
        compiler_params=pltpu.CompilerParams(needs_layout_passes=False),
        scratch_types=[
            pltpu.VMEM_SHARED((ACC_ROWS, EMB_DIM), jnp.float32),
            pltpu.VMEM((2 * BLK + 2 * L,), jnp.int32),
            pltpu.VMEM((BLK + 2 * C,), jnp.int32),
            pltpu.VMEM((C,), jnp.int32),
            pltpu.VMEM((C,), jnp.int32),
            pltpu.VMEM((C,), jnp.int32),
            pltpu.VMEM((C,), jnp.float32),
            pltpu.VMEM((C,), jnp.float32),
            pltpu.VMEM((C, EMB_DIM), jnp.float32),
            pltpu.VMEM((C, EMB_DIM), jnp.float32),
            pltpu.VMEM((LUT_N,), jnp.float32),
            pltpu.SemaphoreType.DMA,
            pltpu.SemaphoreType.DMA,
        ],
    )
    def update_kernel(w_hbm, pairs_hbm, lut_hbm, out_hbm,
                      acc, scanbuf, idxbuf,
                      cdst, csrc, coff, cbias, clablr,
                      drows, srows, lutv, sem0, sem1):
        cid = lax.axis_index("c")
        sid = lax.axis_index("s")

        pltpu.sync_copy(lut_hbm, lutv)

        iota = lax.iota(jnp.int32, L)
        one16 = jnp.full((L,), 1, jnp.int32)
        zero16 = jnp.full((L,), 0, jnp.int32)

        def range_body(rr, _):
            r = rr * NC + cid
            rlo = r * R
            rhi = rlo + R
            rlo16 = zero16 + rlo
            rhi16 = zero16 + rhi
            row0 = rlo + sid * SUB_ROWS

            @pl.when(row0 < NUM_NODES)
            def _():
                pltpu.sync_copy(w_hbm.at[pl.ds(row0, SUB_ROWS)],
                                acc.at[pl.ds(sid * SUB_ROWS, SUB_ROWS)])
            plsc.subcore_barrier()

            even = (iota & one16) == zero16
            scanbuf[pl.ds(2 * BLK, L)] = jnp.where(even, rhi16, zero16)

            def block_body(b, _):
                base_u = sid * S + b * BLK

                pltpu.sync_copy(pairs_hbm.at[pl.ds(2 * base_u, 2 * BLK)],
                                scanbuf.at[pl.ds(0, 2 * BLK)])

                def scan_body(g, cnt):
                    d16 = plsc.load_gather(scanbuf, [iota * 2 + g * (2 * L)])
                    m = (d16 >= rlo16) & (d16 < rhi16)
                    mi = jnp.where(m, one16, zero16)
                    off = cnt + plsc.cumsum(mi) - 1
                    plsc.store_scatter(idxbuf, [off], iota + g * L, mask=m)
                    return cnt + jnp.sum(mi)

                cnt = lax.fori_loop(0, BLK // L, scan_body, jnp.int32(0))

                for k in range(C // L):
                    idxbuf[pl.ds(cnt + k * L, L)] = zero16 + BLK

                def chunk_body(k, _):
                    for j in range(C // L):
                        ids = idxbuf[pl.ds(k * C + j * L, L)]
                        d = plsc.load_gather(scanbuf, [ids * 2])
                        slv = plsc.load_gather(scanbuf, [ids * 2 + 1])
                        lab = slv >> 17
                        pos = lab == one16
                        cdst[pl.ds(j * L, L)] = jnp.minimum(d, NUM_NODES - 1)
                        csrc[pl.ds(j * L, L)] = slv & 0x1FFFF
                        coff[pl.ds(j * L, L)] = d - rlo
                        cbias[pl.ds(j * L, L)] = jnp.where(
                            pos, jnp.full((L,), BIAS_P, jnp.float32),
                            jnp.full((L,), BIAS_N, jnp.float32))
                        clablr[pl.ds(j * L, L)] = jnp.where(
                            pos, jnp.full((L,), LR, jnp.float32),
                            jnp.full((L,), 0.0, jnp.float32))
                    cp0 = pltpu.async_copy(w_hbm.at[cdst], drows, sem0)
                    cp1 = pltpu.async_copy(w_hbm.at[csrc], srows, sem1)
                    cp0.wait()
                    cp1.wait()

                    for g in range(C // L):
                        rows16 = iota + g * L
                        bias16 = cbias[pl.ds(g * L, L)]
                        ll16 = clablr[pl.ds(g * L, L)]

                        def dim_body(j, _, rows16=rows16, bias16=bias16,
                                     ll16=ll16):
                            jj = zero16 + j
                            sv = plsc.load_gather(srows, [rows16, jj])
                            dv = plsc.load_gather(drows, [rows16, jj])
                            score = dv * sv - bias16
                            score = jnp.minimum(jnp.maximum(score, -6.0), 6.0)
                            idx = ((score + 6.01) * 100.0).astype(jnp.int32)
                            sig = plsc.load_gather(lutv, [idx])
                            coef = ll16 - sig * jnp.float32(LR)
                            plsc.store_scatter(srows, [rows16, jj], sv * coef)
                            return 0

                        lax.fori_loop(0, EMB_DIM, dim_body, 0)

                    pltpu.sync_copy(srows, acc.at[coff], add=True)
                    return 0

                nch = (cnt + (C - 1)) // C
                lax.fori_loop(0, nch, chunk_body, 0)
                return 0

            lax.fori_loop(0, S // BLK, block_body, 0)

            plsc.subcore_barrier()

            @pl.when(row0 < NUM_NODES)
            def _():
                pltpu.sync_copy(acc.at[pl.ds(sid * SUB_ROWS, SUB_ROWS)],
                                out_hbm.at[pl.ds(row0, SUB_ROWS)])
            plsc.subcore_barrier()
            return 0

        lax.fori_loop(0, NRANGE // NC, range_body, 0)

    return update_kernel


def kernel(W, u, v, label):
    dst = jnp.concatenate([u, v])
    srclab = jnp.concatenate([v, u]) | (jnp.concatenate([label, label]) << 17)
    pairs = jnp.stack([dst, srclab], axis=1).reshape(-1)
    return _build(dst.shape[0])(W, pairs, _lookup_table())


# speedup vs baseline: 1.0140x; 1.0140x over previous
import functools
from math import log

import jax
import jax.numpy as jnp
from jax import lax
from jax.experimental import pallas as pl
from jax.experimental.pallas import tpu as pltpu, tpu_sc as plsc

NUM_NODES = 100000
EMB_DIM = 128
NEGATIVE = 5
LR = 0.025

NC, NS, L = 2, 16, 16
R = 12800
NRANGE = 8
SUB_ROWS = R // NS
BLK = 2048
C = 64
ACC_ROWS = R + 8
LUT_N = 1216

BIAS_P = float(log(NUM_NODES))
BIAS_N = float(log(NUM_NODES / NEGATIVE))


def _lookup_table():
    t = jax.nn.sigmoid(jnp.arange(-6.01, 6.01, 0.01, dtype=jnp.float32))
    t = t.at[0].set(0.0)
    t = t.at[-1].set(1.0)
    return jnp.concatenate([t, jnp.full((LUT_N - t.shape[0],), 1.0, jnp.float32)])


@functools.lru_cache(maxsize=None)
def _build(total_updates: int):
    S = total_updates // NS

    mesh = plsc.VectorSubcoreMesh(
        core_axis_name="c", subcore_axis_name="s", num_cores=NC, num_subcores=NS
    )

    @functools.partial(
        pl.kernel,
        out_type=jax.ShapeDtypeStruct((NUM_NODES, EMB_DIM), jnp.float32),
        mesh=mesh,
        compiler_params=pltpu.CompilerParams(needs_layout_passes=False),
        scratch_types=[
            pltpu.VMEM_SHARED((ACC_ROWS, EMB_DIM), jnp.float32),
            pltpu.VMEM((2 * BLK + 2 * L,), jnp.int32),
            pltpu.VMEM((BLK + 2 * C,), jnp.int32),
            pltpu.VMEM((2 * C,), jnp.int32),
            pltpu.VMEM((C,), jnp.int32),
            pltpu.VMEM((C,), jnp.float32),
            pltpu.VMEM((C,), jnp.float32),
            pltpu.VMEM((2 * C, EMB_DIM), jnp.float32),
            pltpu.VMEM((LUT_N,), jnp.float32),
            pltpu.SemaphoreType.DMA,
        ],
    )
    def update_kernel(w_hbm, pairs_hbm, lut_hbm, out_hbm,
                      acc, scanbuf, idxbuf,
                      cidx, coff, cbias, clablr,
                      rows, lutv, sem0):
        cid = lax.axis_index("c")
        sid = lax.axis_index("s")

        pltpu.sync_copy(lut_hbm, lutv)

        iota = lax.iota(jnp.int32, L)
        one16 = jnp.full((L,), 1, jnp.int32)
        zero16 = jnp.full((L,), 0, jnp.int32)

        def range_body(rr, _):
            r = rr * NC + cid
            rlo = r * R
            rhi = rlo + R
            rlo16 = zero16 + rlo
            rhi16 = zero16 + rhi
            row0 = rlo + sid * SUB_ROWS

            @pl.when(row0 < NUM_NODES)
            def _():
                pltpu.sync_copy(w_hbm.at[pl.ds(row0, SUB_ROWS)],
                                acc.at[pl.ds(sid * SUB_ROWS, SUB_ROWS)])
            plsc.subcore_barrier()

            even = (iota & one16) == zero16
            scanbuf[pl.ds(2 * BLK, L)] = jnp.where(even, rhi16, zero16)

            def block_body(b, _):
                base_u = sid * S + b * BLK

                pltpu.sync_copy(pairs_hbm.at[pl.ds(2 * base_u, 2 * BLK)],
                                scanbuf.at[pl.ds(0, 2 * BLK)])

                def scan_body(g, cnt):
                    d16 = plsc.load_gather(scanbuf, [iota * 2 + g * (2 * L)])
                    m = (d16 >= rlo16) & (d16 < rhi16)
                    mi = jnp.where(m, one16, zero16)
                    off = cnt + plsc.cumsum(mi) - 1
                    plsc.store_scatter(idxbuf, [off], iota + g * L, mask=m)
                    return cnt + jnp.sum(mi)

                cnt = lax.fori_loop(0, BLK // L, scan_body, jnp.int32(0),
                                    unroll=4)

                for k in range(C // L):
                    idxbuf[pl.ds(cnt + k * L, L)] = zero16 + BLK

                def chunk_body(k, _):
                    for j in range(C // L):
                        ids = idxbuf[pl.ds(k * C + j * L, L)]
                        d = plsc.load_gather(scanbuf, [ids * 2])
                        slv = plsc.load_gather(scanbuf, [ids * 2 + 1])
                        lab = slv >> 17
                        pos = lab == one16
                        cidx[pl.ds(j * L, L)] = jnp.minimum(d, NUM_NODES - 1)
                        cidx[pl.ds(C + j * L, L)] = slv & 0x1FFFF
                        coff[pl.ds(j * L, L)] = d - rlo
                        cbias[pl.ds(j * L, L)] = jnp.where(
                            pos, jnp.full((L,), BIAS_P, jnp.float32),
                            jnp.full((L,), BIAS_N, jnp.float32))
                        clablr[pl.ds(j * L, L)] = jnp.where(
                            pos, jnp.full((L,), LR, jnp.float32),
                            jnp.full((L,), 0.0, jnp.float32))
                    pltpu.async_copy(w_hbm.at[cidx], rows, sem0).wait()

                    for g in range(C // L):
                        rows16 = iota + g * L
                        bias16 = cbias[pl.ds(g * L, L)]
                        ll16 = clablr[pl.ds(g * L, L)]

                        def dim_body(j, _, rows16=rows16, bias16=bias16,
                                     ll16=ll16):
                            jj = zero16 + j
                            sv = plsc.load_gather(rows, [rows16 + C, jj])
                            dv = plsc.load_gather(rows, [rows16, jj])
                            score = dv * sv - bias16
                            score = jnp.minimum(jnp.maximum(score, -6.0), 6.0)
                            idx = ((score + 6.01) * 100.0).astype(jnp.int32)
                            sig = plsc.load_gather(lutv, [idx])
                            coef = ll16 - sig * jnp.float32(LR)
                            plsc.store_scatter(rows, [rows16 + C, jj], sv * coef)
                            return 0

                        lax.fori_loop(0, EMB_DIM, dim_body, 0, unroll=8)

                    pltpu.sync_copy(rows.at[pl.ds(C, C)], acc.at[coff], add=True)
                    return 0

                nch = (cnt + (C - 1)) // C
                lax.fori_loop(0, nch, chunk_body, 0)
                return 0

            lax.fori_loop(0, S // BLK, block_body, 0)

            plsc.subcore_barrier()

            @pl.when(row0 < NUM_NODES)
            def _():
                pltpu.sync_copy(acc.at[pl.ds(sid * SUB_ROWS, SUB_ROWS)],
                                out_hbm.at[pl.ds(row0, SUB_ROWS)])
            plsc.subcore_barrier()
            return 0

        lax.fori_loop(0, NRANGE // NC, range_body, 0)

    return update_kernel


def kernel(W, u, v, label):
    dst = jnp.concatenate([u, v])
    srclab = jnp.concatenate([v, u]) | (jnp.concatenate([label, label]) << 17)
    pairs = jnp.stack([dst, srclab], axis=1).reshape(-1)
    return _build(dst.shape[0])(W, pairs, _lookup_table())


# speedup vs baseline: 1.1228x; 1.1072x over previous
import functools
from math import log

import jax
import jax.numpy as jnp
from jax import lax
from jax.experimental import pallas as pl
from jax.experimental.pallas import tpu as pltpu, tpu_sc as plsc

NUM_NODES = 100000
EMB_DIM = 128
NEGATIVE = 5
LR = 0.025

NC, NS, L = 2, 16, 16
R = 10000
NRANGE = 10
INIT_CHUNK = 200
N_INIT = R // INIT_CHUNK
BLK = 2048
SCAN_W = 2 * BLK + 2 * L
C = 48
ACC_ROWS = R + 8
LUT_N = 1216

BIAS_P = float(log(NUM_NODES))
BIAS_N = float(log(NUM_NODES / NEGATIVE))


def _lookup_table():
    t = jax.nn.sigmoid(jnp.arange(-6.01, 6.01, 0.01, dtype=jnp.float32))
    t = t.at[0].set(0.0)
    t = t.at[-1].set(1.0)
    return jnp.concatenate([t, jnp.full((LUT_N - t.shape[0],), 1.0, jnp.float32)])


@functools.lru_cache(maxsize=None)
def _build(total_updates: int):
    S = total_updates // NS

    mesh = plsc.VectorSubcoreMesh(
        core_axis_name="c", subcore_axis_name="s", num_cores=NC, num_subcores=NS
    )

    @functools.partial(
        pl.kernel,
        out_type=jax.ShapeDtypeStruct((NUM_NODES, EMB_DIM), jnp.float32),
        mesh=mesh,
        compiler_params=pltpu.CompilerParams(needs_layout_passes=False),
        scratch_types=[
            pltpu.VMEM_SHARED((ACC_ROWS, EMB_DIM), jnp.float32),
            pltpu.VMEM((2 * SCAN_W,), jnp.int32),
            pltpu.VMEM((BLK + 2 * C + L,), jnp.int32),
            pltpu.VMEM((2, 2 * C), jnp.int32),
            pltpu.VMEM((2, C), jnp.int32),
            pltpu.VMEM((2, C), jnp.float32),
            pltpu.VMEM((2, C), jnp.float32),
            pltpu.VMEM((2, 2 * C, EMB_DIM), jnp.float32),
            pltpu.VMEM((C, EMB_DIM), jnp.float32),
            pltpu.VMEM((LUT_N,), jnp.float32),
            pltpu.SemaphoreType.DMA,
            pltpu.SemaphoreType.DMA,
            pltpu.SemaphoreType.DMA,
        ],
    )
    def update_kernel(w_hbm, pairs_hbm, lut_hbm, out_hbm,
                      acc, scanbuf, idxbuf,
                      cidx, coff, cbias, clablr,
                      rows, dbuf, lutv, sem_s, sem_g0, sem_g1):
        cid = lax.axis_index("c")
        sid = lax.axis_index("s")

        pltpu.sync_copy(lut_hbm, lutv)

        iota = lax.iota(jnp.int32, L)
        one16 = jnp.full((L,), 1, jnp.int32)
        zero16 = jnp.full((L,), 0, jnp.int32)
        gsems = (sem_g0, sem_g1)

        def range_body(rr, _):
            r = rr * NC + cid
            rlo = r * R
            rhi = rlo + R
            rlo16 = zero16 + rlo
            rhi16 = zero16 + rhi

            for t in range(-(-N_INIT // NS)):
                cix = t * NS + sid

                @pl.when(cix < N_INIT)
                def _(cix=cix):
                    pltpu.sync_copy(
                        w_hbm.at[pl.ds(rlo + cix * INIT_CHUNK, INIT_CHUNK)],
                        acc.at[pl.ds(cix * INIT_CHUNK, INIT_CHUNK)])
            plsc.subcore_barrier()

            even = (iota & one16) == zero16
            sent = jnp.where(even, rhi16, zero16)
            scanbuf[pl.ds(2 * BLK, L)] = sent
            scanbuf[pl.ds(SCAN_W + 2 * BLK, L)] = sent

            pltpu.async_copy(pairs_hbm.at[pl.ds(2 * sid * S, 2 * BLK)],
                             scanbuf.at[pl.ds(0, 2 * BLK)], sem_s)

            def block_body(b, _):
                pb = b & 1
                sb = pb * SCAN_W

                pltpu.make_async_copy(
                    pairs_hbm.at[pl.ds(0, 2 * BLK)],
                    scanbuf.at[pl.ds(0, 2 * BLK)], sem_s).wait()

                @pl.when(b + 1 < S // BLK)
                def _():
                    nb = (1 - pb) * SCAN_W
                    base_n = sid * S + (b + 1) * BLK
                    pltpu.async_copy(pairs_hbm.at[pl.ds(2 * base_n, 2 * BLK)],
                                     scanbuf.at[pl.ds(nb, 2 * BLK)], sem_s)

                def scan_body(g, cnt):
                    d16 = plsc.load_gather(scanbuf,
                                           [iota * 2 + (sb + g * (2 * L))])
                    m = (d16 >= rlo16) & (d16 < rhi16)
                    mi = jnp.where(m, one16, zero16)
                    off = cnt + plsc.cumsum(mi) - 1
                    plsc.store_scatter(idxbuf, [off], iota + g * L, mask=m)
                    return cnt + jnp.sum(mi)

                cnt = lax.fori_loop(0, BLK // L, scan_body, jnp.int32(0),
                                    unroll=4)

                for k in range(C // L):
                    idxbuf[pl.ds(cnt + k * L, L)] = zero16 + BLK

                nch = (cnt + (C - 1)) // C

                def issue(k, p):
                    for j in range(C // L):
                        ids = idxbuf[pl.ds(k * C + j * L, L)]
                        d = plsc.load_gather(scanbuf, [ids * 2 + sb])
                        slv = plsc.load_gather(scanbuf, [ids * 2 + (sb + 1)])
                        pos = (slv >> 17) == one16
                        cidx[p, pl.ds(j * L, L)] = jnp.minimum(d, NUM_NODES - 1)
                        cidx[p, pl.ds(C + j * L, L)] = slv & 0x1FFFF
                        coff[p, pl.ds(j * L, L)] = d - rlo
                        cbias[p, pl.ds(j * L, L)] = jnp.where(
                            pos, jnp.full((L,), BIAS_P, jnp.float32),
                            jnp.full((L,), BIAS_N, jnp.float32))
                        clablr[p, pl.ds(j * L, L)] = jnp.where(
                            pos, jnp.full((L,), LR, jnp.float32),
                            jnp.full((L,), 0.0, jnp.float32))
                    pltpu.async_copy(w_hbm.at[cidx.at[p]], rows.at[p],
                                     gsems[p])

                def consume(p):
                    pltpu.make_async_copy(w_hbm.at[cidx.at[p]], rows.at[p],
                                          gsems[p]).wait()
                    rp = rows.at[p]
                    for g in range(C // L):
                        rows16 = iota + g * L
                        bias16 = cbias[p, pl.ds(g * L, L)]
                        ll16 = clablr[p, pl.ds(g * L, L)]

                        def dim_body(j, _, rp=rp, rows16=rows16,
                                     bias16=bias16, ll16=ll16):
                            jj = zero16 + j
                            sv = plsc.load_gather(rp, [rows16 + C, jj])
                            dv = plsc.load_gather(rp, [rows16, jj])
                            score = dv * sv - bias16
                            score = jnp.minimum(jnp.maximum(score, -6.0), 6.0)
                            idx = ((score + 6.01) * 100.0).astype(jnp.int32)
                            sig = plsc.load_gather(lutv, [idx])
                            coef = ll16 - sig * jnp.float32(LR)
                            plsc.store_scatter(dbuf, [rows16, jj], sv * coef)
                            return 0

                        lax.fori_loop(0, EMB_DIM, dim_body, 0, unroll=8)

                    pltpu.sync_copy(dbuf, acc.at[coff.at[p]], add=True)

                @pl.when(nch >= 1)
                def _():
                    issue(jnp.int32(0), 0)

                @pl.when(nch >= 2)
                def _():
                    issue(jnp.int32(1), 1)

                def chunk_pair(m, _):
                    for p in range(2):
                        k = 2 * m + p

                        @pl.when(k < nch)
                        def _(k=k, p=p):
                            consume(p)

                        @pl.when(k + 2 < nch)
                        def _(k=k, p=p):
                            issue(k + 2, p)
                    return 0

                lax.fori_loop(0, (nch + 1) // 2, chunk_pair, 0)
                return 0

            lax.fori_loop(0, S // BLK, block_body, 0)

            plsc.subcore_barrier()
            for t in range(-(-N_INIT // NS)):
                cix = t * NS + sid

                @pl.when(cix < N_INIT)
                def _(cix=cix):
                    pltpu.sync_copy(
                        acc.at[pl.ds(cix * INIT_CHUNK, INIT_CHUNK)],
                        out_hbm.at[pl.ds(rlo + cix * INIT_CHUNK, INIT_CHUNK)])
            plsc.subcore_barrier()
            return 0

        lax.fori_loop(0, NRANGE // NC, range_body, 0)

    return update_kernel


def kernel(W, u, v, label):
    dst = jnp.concatenate([u, v])
    srclab = jnp.concatenate([v, u]) | (jnp.concatenate([label, label]) << 17)
    pairs = jnp.stack([dst, srclab], axis=1).reshape(-1)
    return _build(dst.shape[0])(W, pairs, _lookup_table())


# speedup vs baseline: 2.3383x; 2.0825x over previous
import functools
from math import log

import jax
import jax.numpy as jnp
from jax import lax
from jax.experimental import pallas as pl
from jax.experimental.pallas import tpu as pltpu, tpu_sc as plsc

NUM_NODES = 100000
EMB_DIM = 128
NEGATIVE = 5
LR = 0.025

NC, NS, L = 2, 16, 16
R = 10000
NRANGE = 10
INIT_CHUNK = 200
N_INIT = R // INIT_CHUNK
BLK = 2048
SCAN_W = 2 * BLK + 2 * L
C = 48
ACC_ROWS = R + 8
LUT_N = 1216

BIAS_P = float(log(NUM_NODES))
BIAS_N = float(log(NUM_NODES / NEGATIVE))


def _lookup_table():
    t = jax.nn.sigmoid(jnp.arange(-6.01, 6.01, 0.01, dtype=jnp.float32))
    t = t.at[0].set(0.0)
    t = t.at[-1].set(1.0)
    return jnp.concatenate([t, jnp.full((LUT_N - t.shape[0],), 1.0, jnp.float32)])


@functools.lru_cache(maxsize=None)
def _build(total_updates: int):
    S = total_updates // NS

    mesh = plsc.VectorSubcoreMesh(
        core_axis_name="c", subcore_axis_name="s", num_cores=NC, num_subcores=NS
    )

    @functools.partial(
        pl.kernel,
        out_type=jax.ShapeDtypeStruct((NUM_NODES, EMB_DIM), jnp.float32),
        mesh=mesh,
        compiler_params=pltpu.CompilerParams(needs_layout_passes=False),
        scratch_types=[
            pltpu.VMEM_SHARED((ACC_ROWS, EMB_DIM), jnp.float32),
            pltpu.VMEM((2 * SCAN_W,), jnp.int32),
            pltpu.VMEM((BLK + 2 * C + L,), jnp.int32),
            pltpu.VMEM((2, 2 * C), jnp.int32),
            pltpu.VMEM((2, C), jnp.int32),
            pltpu.VMEM((2, C), jnp.float32),
            pltpu.VMEM((2, C), jnp.float32),
            pltpu.VMEM((2, 2 * C, EMB_DIM), jnp.float32),
            pltpu.VMEM((C, EMB_DIM), jnp.float32),
            pltpu.VMEM((LUT_N,), jnp.float32),
            pltpu.SemaphoreType.DMA,
            pltpu.SemaphoreType.DMA,
            pltpu.SemaphoreType.DMA,
        ],
    )
    def update_kernel(w_hbm, pairs_hbm, lut_hbm, out_hbm,
                      acc, scanbuf, idxbuf,
                      cidx, coff, cbias, clablr,
                      rows, dbuf, lutv, sem_s, sem_g0, sem_g1):
        cid = lax.axis_index("c")
        sid = lax.axis_index("s")

        pltpu.sync_copy(lut_hbm, lutv)

        iota = lax.iota(jnp.int32, L)
        one16 = jnp.full((L,), 1, jnp.int32)
        zero16 = jnp.full((L,), 0, jnp.int32)
        gsems = (sem_g0, sem_g1)

        def range_body(rr, _):
            r = rr * NC + cid
            rlo = r * R
            rhi = rlo + R
            rlo16 = zero16 + rlo
            rhi16 = zero16 + rhi

            for t in range(-(-N_INIT // NS)):
                cix = t * NS + sid

                @pl.when(cix < N_INIT)
                def _(cix=cix):
                    pltpu.sync_copy(
                        w_hbm.at[pl.ds(rlo + cix * INIT_CHUNK, INIT_CHUNK)],
                        acc.at[pl.ds(cix * INIT_CHUNK, INIT_CHUNK)])
            plsc.subcore_barrier()

            even = (iota & one16) == zero16
            sent = jnp.where(even, rhi16, zero16)
            scanbuf[pl.ds(2 * BLK, L)] = sent
            scanbuf[pl.ds(SCAN_W + 2 * BLK, L)] = sent

            pltpu.async_copy(pairs_hbm.at[pl.ds(2 * sid * S, 2 * BLK)],
                             scanbuf.at[pl.ds(0, 2 * BLK)], sem_s)

            def block_body(b, _):
                pb = b & 1
                sb = pb * SCAN_W

                pltpu.make_async_copy(
                    pairs_hbm.at[pl.ds(0, 2 * BLK)],
                    scanbuf.at[pl.ds(0, 2 * BLK)], sem_s).wait()

                @pl.when(b + 1 < S // BLK)
                def _():
                    nb = (1 - pb) * SCAN_W
                    base_n = sid * S + (b + 1) * BLK
                    pltpu.async_copy(pairs_hbm.at[pl.ds(2 * base_n, 2 * BLK)],
                                     scanbuf.at[pl.ds(nb, 2 * BLK)], sem_s)

                def scan_body(g, cnt):
                    d16 = plsc.load_gather(scanbuf,
                                           [iota * 2 + (sb + g * (2 * L))])
                    m = (d16 >= rlo16) & (d16 < rhi16)
                    mi = jnp.where(m, one16, zero16)
                    off = cnt + plsc.cumsum(mi) - 1
                    plsc.store_scatter(idxbuf, [off], iota + g * L, mask=m)
                    return cnt + jnp.sum(mi)

                cnt = lax.fori_loop(0, BLK // L, scan_body, jnp.int32(0),
                                    unroll=4)

                for k in range(C // L):
                    idxbuf[pl.ds(cnt + k * L, L)] = zero16 + BLK

                nch = (cnt + (C - 1)) // C

                def issue(k, p):
                    for j in range(C // L):
                        ids = idxbuf[pl.ds(k * C + j * L, L)]
                        d = plsc.load_gather(scanbuf, [ids * 2 + sb])
                        slv = plsc.load_gather(scanbuf, [ids * 2 + (sb + 1)])
                        pos = (slv >> 17) == one16
                        cidx[p, pl.ds(j * L, L)] = jnp.minimum(d, NUM_NODES - 1)
                        cidx[p, pl.ds(C + j * L, L)] = slv & 0x1FFFF
                        coff[p, pl.ds(j * L, L)] = d - rlo
                        cbias[p, pl.ds(j * L, L)] = jnp.where(
                            pos, jnp.full((L,), BIAS_P, jnp.float32),
                            jnp.full((L,), BIAS_N, jnp.float32))
                        clablr[p, pl.ds(j * L, L)] = jnp.where(
                            pos, jnp.full((L,), LR, jnp.float32),
                            jnp.full((L,), 0.0, jnp.float32))
                    pltpu.async_copy(w_hbm.at[cidx.at[p]], rows.at[p],
                                     gsems[p])

                def consume(p):
                    pltpu.make_async_copy(w_hbm.at[cidx.at[p]], rows.at[p],
                                          gsems[p]).wait()
                    rp = rows.at[p]

                    def upd_body(u, _, rp=rp):
                        uu = zero16 + u
                        bias16 = plsc.load_gather(cbias.at[p], [uu])
                        ll16 = plsc.load_gather(clablr.at[p], [uu])
                        for j in range(EMB_DIM // L):
                            sv = rp[u + C, pl.ds(j * L, L)]
                            dv = rp[u, pl.ds(j * L, L)]
                            score = dv * sv - bias16
                            score = jnp.minimum(jnp.maximum(score, -6.0), 6.0)
                            idx = ((score + 6.01) * 100.0).astype(jnp.int32)
                            sig = plsc.load_gather(lutv, [idx])
                            coef = ll16 - sig * jnp.float32(LR)
                            dbuf[u, pl.ds(j * L, L)] = sv * coef
                        return 0

                    lax.fori_loop(0, C, upd_body, 0)

                    pltpu.sync_copy(dbuf, acc.at[coff.at[p]], add=True)

                @pl.when(nch >= 1)
                def _():
                    issue(jnp.int32(0), 0)

                @pl.when(nch >= 2)
                def _():
                    issue(jnp.int32(1), 1)

                def chunk_pair(m, _):
                    for p in range(2):
                        k = 2 * m + p

                        @pl.when(k < nch)
                        def _(k=k, p=p):
                            consume(p)

                        @pl.when(k + 2 < nch)
                        def _(k=k, p=p):
                            issue(k + 2, p)
                    return 0

                lax.fori_loop(0, (nch + 1) // 2, chunk_pair, 0)
                return 0

            lax.fori_loop(0, S // BLK, block_body, 0)

            plsc.subcore_barrier()
            for t in range(-(-N_INIT // NS)):
                cix = t * NS + sid

                @pl.when(cix < N_INIT)
                def _(cix=cix):
                    pltpu.sync_copy(
                        acc.at[pl.ds(cix * INIT_CHUNK, INIT_CHUNK)],
                        out_hbm.at[pl.ds(rlo + cix * INIT_CHUNK, INIT_CHUNK)])
            plsc.subcore_barrier()
            return 0

        lax.fori_loop(0, NRANGE // NC, range_body, 0)

    return update_kernel


def kernel(W, u, v, label):
    dst = jnp.concatenate([u, v])
    srclab = jnp.concatenate([v, u]) | (jnp.concatenate([label, label]) << 17)
    pairs = jnp.stack([dst, srclab], axis=1).reshape(-1)
    return _build(dst.shape[0])(W, pairs, _lookup_table())
